# Initial kernel scaffold; baseline (speedup 1.0000x reference)
import jax
import jax.numpy as jnp
from jax.experimental import pallas as pl


def kernel(hidden_states, expert_outputs, gate_w, gate_b):
    raise NotImplementedError("write your pallas kernel here")


# speedup vs baseline: 1.7175x; 1.7175x over previous
import functools

import jax
import jax.numpy as jnp
from jax import lax
from jax.experimental import pallas as pl
from jax.experimental.pallas import tpu as pltpu
from jax.experimental.pallas import tpu_sc as plsc

E = 8
N = 4096
D = 2048
K = 2
L = 16

NC = 2
NS = 16
NW = NC * NS
TPW = N // NW
T = 8
NSUB = TPW // T
ROWS = K * T

TOK_BLK = 256


def _gating_body(h_ref, gw_ref, gb_ref, wb_ref, fi_ref):
    blk = pl.program_id(0)
    h = h_ref[...]
    gw = gw_ref[...]
    gb = gb_ref[...]
    scores = lax.dot_general(h, gw, (((1,), (1,)), ((), ())),
                             preferred_element_type=jnp.float32) + gb
    iota_e = lax.broadcasted_iota(jnp.int32, (TOK_BLK, E), 1)
    m0 = jnp.max(scores, axis=1, keepdims=True)
    a0 = jnp.min(jnp.where(scores == m0, iota_e, E), axis=1, keepdims=True)
    masked = jnp.where(iota_e == a0, -jnp.inf, scores)
    m1 = jnp.max(masked, axis=1, keepdims=True)
    a1 = jnp.min(jnp.where(masked == m1, iota_e, E), axis=1, keepdims=True)
    w0 = 1.0 / (1.0 + jnp.exp(m1 - m0))
    w1 = 1.0 - w0
    lane = lax.broadcasted_iota(jnp.int32, (TOK_BLK, 2 * L), 1)
    wb_ref[...] = jnp.where(lane < L, w0, w1)
    tok = blk * TOK_BLK + lax.broadcasted_iota(jnp.int32, (TOK_BLK, K), 0)
    which = lax.broadcasted_iota(jnp.int32, (TOK_BLK, K), 1)
    fi_ref[...] = jnp.where(which == 0, a0, a1) * N + tok


def _gating(hidden_states, gate_w, gate_b2d):
    return pl.pallas_call(
        _gating_body,
        grid=(N // TOK_BLK,),
        in_specs=[
            pl.BlockSpec((TOK_BLK, D), lambda i: (i, 0)),
            pl.BlockSpec((E, D), lambda i: (0, 0)),
            pl.BlockSpec((1, E), lambda i: (0, 0)),
        ],
        out_specs=[
            pl.BlockSpec((TOK_BLK, 2 * L), lambda i: (i, 0)),
            pl.BlockSpec((TOK_BLK, K), lambda i: (i, 0)),
        ],
        out_shape=[
            jax.ShapeDtypeStruct((N, 2 * L), jnp.float32),
            jax.ShapeDtypeStruct((N, K), jnp.int32),
        ],
    )(hidden_states, gate_w, gate_b2d)


def _combine_body(eo_ref, idx_ref, w_ref, out_ref,
                  idx_v, w_v, r0, r1, o0, o1, g0, g1, s0, s1):
    wid = lax.axis_index("s") * NC + lax.axis_index("c")
    tok0 = wid * TPW
    pltpu.sync_copy(idx_ref.at[pl.ds(tok0 * K, TPW * K)], idx_v)
    pltpu.sync_copy(w_ref.at[pl.ds(tok0 * 2 * L, TPW * 2 * L)], w_v)

    rbufs = (r0, r1)
    obufs = (o0, o1)
    gsems = (g0, g1)
    ssems = (s0, s1)

    def start_gather(j, b):
        return pltpu.async_copy(
            eo_ref.at[idx_v.at[pl.ds(j * ROWS, ROWS)]], rbufs[b], gsems[b])

    gathers = {0: start_gather(0, 0)}
    stores = {}
    for j in range(NSUB):
        cur = j & 1
        if j + 1 < NSUB:
            gathers[j + 1] = start_gather(j + 1, 1 - cur)
        gathers[j].wait()
        if j >= 2:
            stores[j - 2].wait()
        rbuf = rbufs[cur]
        obuf = obufs[cur]

        def tok_body(t, _, jj=j, rbuf=rbuf, obuf=obuf):
            woff = (jj * T + t) * (2 * L)
            wv0 = w_v[pl.ds(woff, L)]
            wv1 = w_v[pl.ds(woff + L, L)]

            def d_body(d, _):
                base = d * (4 * L)
                for u in range(4):
                    c = base + u * L
                    obuf[t, pl.ds(c, L)] = (
                        rbuf[2 * t, pl.ds(c, L)] * wv0
                        + rbuf[2 * t + 1, pl.ds(c, L)] * wv1)
                return 0

            lax.fori_loop(0, D // (4 * L), d_body, 0)
            return 0

        lax.fori_loop(0, T, tok_body, 0)
        stores[j] = pltpu.async_copy(
            obuf, out_ref.at[pl.ds(tok0 + j * T, T)], ssems[cur])
    stores[NSUB - 2].wait()
    stores[NSUB - 1].wait()


@functools.cache
def _make_combine():
    return pl.kernel(
        _combine_body,
        out_type=jax.ShapeDtypeStruct((N, D), jnp.float32),
        mesh=plsc.VectorSubcoreMesh(core_axis_name="c", subcore_axis_name="s",
                                    num_cores=NC, num_subcores=NS),
        scratch_types=[
            pltpu.VMEM((TPW * K,), jnp.int32),
            pltpu.VMEM((TPW * 2 * L,), jnp.float32),
            pltpu.VMEM((ROWS, D), jnp.float32),
            pltpu.VMEM((ROWS, D), jnp.float32),
            pltpu.VMEM((T, D), jnp.float32),
            pltpu.VMEM((T, D), jnp.float32),
            pltpu.SemaphoreType.DMA,
            pltpu.SemaphoreType.DMA,
            pltpu.SemaphoreType.DMA,
            pltpu.SemaphoreType.DMA,
        ],
    )


def kernel(hidden_states, expert_outputs, gate_w, gate_b):
    wb, fi = _gating(hidden_states, gate_w, gate_b.reshape(1, E))
    eo_flat = expert_outputs.reshape(E * N, D)
    return _make_combine()(eo_flat, fi.reshape(N * K), wb.reshape(N * 2 * L))


# speedup vs baseline: 3.2160x; 1.8725x over previous
import functools

import jax
import jax.numpy as jnp
from jax import lax
from jax.experimental import pallas as pl
from jax.experimental.pallas import tpu as pltpu
from jax.experimental.pallas import tpu_sc as plsc

E = 8
N = 4096
D = 2048
K = 2
L = 16

NC = 2
NS = 16
NW = NC * NS
TPW = N // NW
T = 8
NSUB = TPW // T
ROWS = K * T

TOK_BLK = 256


def _gating_body(h_ref, gw_ref, gb_ref, wb_ref, fi_ref):
    blk = pl.program_id(0)
    h = h_ref[...]
    gw = gw_ref[...]
    gb = gb_ref[...]
    scores = lax.dot_general(h, gw, (((1,), (1,)), ((), ())),
                             preferred_element_type=jnp.float32) + gb
    iota_e = lax.broadcasted_iota(jnp.int32, (TOK_BLK, E), 1)
    m0 = jnp.max(scores, axis=1, keepdims=True)
    a0 = jnp.min(jnp.where(scores == m0, iota_e, E), axis=1, keepdims=True)
    masked = jnp.where(iota_e == a0, -jnp.inf, scores)
    m1 = jnp.max(masked, axis=1, keepdims=True)
    a1 = jnp.min(jnp.where(masked == m1, iota_e, E), axis=1, keepdims=True)
    w0 = 1.0 / (1.0 + jnp.exp(m1 - m0))
    w1 = 1.0 - w0
    lane = lax.broadcasted_iota(jnp.int32, (TOK_BLK, 2 * L), 1)
    wb_ref[...] = jnp.where(lane < L, w0, w1)
    tok = blk * TOK_BLK + lax.broadcasted_iota(jnp.int32, (TOK_BLK, K), 0)
    which = lax.broadcasted_iota(jnp.int32, (TOK_BLK, K), 1)
    fi_ref[...] = jnp.where(which == 0, a0, a1) * N + tok


def _gating(hidden_states, gate_w, gate_b2d):
    return pl.pallas_call(
        _gating_body,
        grid=(N // TOK_BLK,),
        in_specs=[
            pl.BlockSpec((TOK_BLK, D), lambda i: (i, 0)),
            pl.BlockSpec((E, D), lambda i: (0, 0)),
            pl.BlockSpec((1, E), lambda i: (0, 0)),
        ],
        out_specs=[
            pl.BlockSpec((TOK_BLK, 2 * L), lambda i: (i, 0)),
            pl.BlockSpec((TOK_BLK, K), lambda i: (i, 0)),
        ],
        out_shape=[
            jax.ShapeDtypeStruct((N, 2 * L), jnp.float32),
            jax.ShapeDtypeStruct((N, K), jnp.int32),
        ],
    )(hidden_states, gate_w, gate_b2d)


def _combine_body(eo_ref, idx_ref, w_ref, out_ref,
                  idx_v, w_v, r0, r1, o0, o1, g0, g1, s0, s1):
    wid = lax.axis_index("s") * NC + lax.axis_index("c")
    tok0 = wid * TPW
    pltpu.sync_copy(idx_ref.at[pl.ds(tok0 * K, TPW * K)], idx_v)
    pltpu.sync_copy(w_ref.at[pl.ds(tok0 * 2 * L, TPW * 2 * L)], w_v)

    rbufs = (r0, r1)
    obufs = (o0, o1)
    gsems = (g0, g1)
    ssems = (s0, s1)

    def start_gather(j, b):
        return pltpu.async_copy(
            eo_ref.at[idx_v.at[pl.ds(j * ROWS, ROWS)]], rbufs[b], gsems[b])

    gathers = {0: start_gather(0, 0)}
    stores = {}
    for j in range(NSUB):
        cur = j & 1
        if j + 1 < NSUB:
            gathers[j + 1] = start_gather(j + 1, 1 - cur)
        gathers[j].wait()
        if j >= 2:
            stores[j - 2].wait()
        rbuf = rbufs[cur]
        obuf = obufs[cur]

        def tok_body(t, _, jj=j, rbuf=rbuf, obuf=obuf):
            woff = (jj * T + t) * (2 * L)
            wv0 = w_v[pl.ds(woff, L)]
            wv1 = w_v[pl.ds(woff + L, L)]

            @plsc.parallel_loop(0, D, L, unroll=8)
            def d_body(c):
                obuf[t, pl.ds(c, L)] = (
                    rbuf[2 * t, pl.ds(c, L)] * wv0
                    + rbuf[2 * t + 1, pl.ds(c, L)] * wv1)

            return 0

        lax.fori_loop(0, T, tok_body, 0)
        stores[j] = pltpu.async_copy(
            obuf, out_ref.at[pl.ds(tok0 + j * T, T)], ssems[cur])
    stores[NSUB - 2].wait()
    stores[NSUB - 1].wait()


@functools.cache
def _make_combine():
    return pl.kernel(
        _combine_body,
        out_type=jax.ShapeDtypeStruct((N, D), jnp.float32),
        mesh=plsc.VectorSubcoreMesh(core_axis_name="c", subcore_axis_name="s",
                                    num_cores=NC, num_subcores=NS),
        scratch_types=[
            pltpu.VMEM((TPW * K,), jnp.int32),
            pltpu.VMEM((TPW * 2 * L,), jnp.float32),
            pltpu.VMEM((ROWS, D), jnp.float32),
            pltpu.VMEM((ROWS, D), jnp.float32),
            pltpu.VMEM((T, D), jnp.float32),
            pltpu.VMEM((T, D), jnp.float32),
            pltpu.SemaphoreType.DMA,
            pltpu.SemaphoreType.DMA,
            pltpu.SemaphoreType.DMA,
            pltpu.SemaphoreType.DMA,
        ],
    )


def kernel(hidden_states, expert_outputs, gate_w, gate_b):
    wb, fi = _gating(hidden_states, gate_w, gate_b.reshape(1, E))
    eo_flat = expert_outputs.reshape(E * N, D)
    return _make_combine()(eo_flat, fi.reshape(N * K), wb.reshape(N * 2 * L))


# speedup vs baseline: 3.3213x; 1.0327x over previous
import functools

import jax
import jax.numpy as jnp
from jax import lax
from jax.experimental import pallas as pl
from jax.experimental.pallas import tpu as pltpu
from jax.experimental.pallas import tpu_sc as plsc

E = 8
N = 4096
D = 2048
K = 2
L = 16

NC = 2
NS = 16
NW = NC * NS
TPW = N // NW
T = 8
NSUB = TPW // T
ROWS = K * T

TOK_BLK = 512


def _gating_body(h_ref, gw_ref, gb_ref, wb_ref, fi_ref):
    blk = pl.program_id(0)
    h = h_ref[...]
    gw = gw_ref[...]
    gb = gb_ref[...]
    scores = lax.dot_general(h, gw, (((1,), (0,)), ((), ())),
                             preferred_element_type=jnp.float32) + gb
    iota_e = lax.broadcasted_iota(jnp.int32, (TOK_BLK, E), 1)
    m0 = jnp.max(scores, axis=1, keepdims=True)
    a0 = jnp.min(jnp.where(scores == m0, iota_e, E), axis=1, keepdims=True)
    masked = jnp.where(iota_e == a0, -jnp.inf, scores)
    m1 = jnp.max(masked, axis=1, keepdims=True)
    a1 = jnp.min(jnp.where(masked == m1, iota_e, E), axis=1, keepdims=True)
    w0 = 1.0 / (1.0 + jnp.exp(m1 - m0))
    w1 = 1.0 - w0
    lane = lax.broadcasted_iota(jnp.int32, (TOK_BLK, 2 * L), 1)
    wb_ref[...] = jnp.where(lane < L, w0, w1)
    tok = blk * TOK_BLK + lax.broadcasted_iota(jnp.int32, (TOK_BLK, K), 0)
    which = lax.broadcasted_iota(jnp.int32, (TOK_BLK, K), 1)
    fi_ref[...] = jnp.where(which == 0, a0, a1) * N + tok


def _gating(hidden_states, gate_w, gate_b2d):
    return pl.pallas_call(
        _gating_body,
        grid=(N // TOK_BLK,),
        in_specs=[
            pl.BlockSpec((TOK_BLK, D), lambda i: (i, 0)),
            pl.BlockSpec((D, E), lambda i: (0, 0)),
            pl.BlockSpec((1, E), lambda i: (0, 0)),
        ],
        out_specs=[
            pl.BlockSpec((TOK_BLK, 2 * L), lambda i: (i, 0)),
            pl.BlockSpec((TOK_BLK, K), lambda i: (i, 0)),
        ],
        out_shape=[
            jax.ShapeDtypeStruct((N, 2 * L), jnp.float32),
            jax.ShapeDtypeStruct((N, K), jnp.int32),
        ],
    )(hidden_states, gate_w, gate_b2d)


def _combine_body(eo_ref, idx_ref, w_ref, out_ref,
                  idx_v, w_v, r0, r1, o0, o1, g0, g1, s0, s1):
    wid = lax.axis_index("s") * NC + lax.axis_index("c")
    tok0 = wid * TPW
    pltpu.sync_copy(idx_ref.at[pl.ds(tok0 * K, TPW * K)], idx_v)
    pltpu.sync_copy(w_ref.at[pl.ds(tok0 * 2 * L, TPW * 2 * L)], w_v)

    rbufs = (r0, r1)
    obufs = (o0, o1)
    gsems = (g0, g1)
    ssems = (s0, s1)

    def start_gather(j, b):
        return pltpu.async_copy(
            eo_ref.at[idx_v.at[pl.ds(j * ROWS, ROWS)]], rbufs[b], gsems[b])

    gathers = {0: start_gather(0, 0)}
    stores = {}
    for j in range(NSUB):
        cur = j & 1
        if j + 1 < NSUB:
            gathers[j + 1] = start_gather(j + 1, 1 - cur)
        gathers[j].wait()
        if j >= 2:
            stores[j - 2].wait()
        rbuf = rbufs[cur]
        obuf = obufs[cur]

        def tok_body(t, _, jj=j, rbuf=rbuf, obuf=obuf):
            woff = (jj * T + t) * (2 * L)
            wv0 = w_v[pl.ds(woff, L)]
            wv1 = w_v[pl.ds(woff + L, L)]

            @plsc.parallel_loop(0, D, L, unroll=8)
            def d_body(c):
                obuf[t, pl.ds(c, L)] = (
                    rbuf[2 * t, pl.ds(c, L)] * wv0
                    + rbuf[2 * t + 1, pl.ds(c, L)] * wv1)

            return 0

        lax.fori_loop(0, T, tok_body, 0)
        stores[j] = pltpu.async_copy(
            obuf, out_ref.at[pl.ds(tok0 + j * T, T)], ssems[cur])
    stores[NSUB - 2].wait()
    stores[NSUB - 1].wait()


@functools.cache
def _make_combine():
    return pl.kernel(
        _combine_body,
        out_type=jax.ShapeDtypeStruct((N, D), jnp.float32),
        mesh=plsc.VectorSubcoreMesh(core_axis_name="c", subcore_axis_name="s",
                                    num_cores=NC, num_subcores=NS),
        scratch_types=[
            pltpu.VMEM((TPW * K,), jnp.int32),
            pltpu.VMEM((TPW * 2 * L,), jnp.float32),
            pltpu.VMEM((ROWS, D), jnp.float32),
            pltpu.VMEM((ROWS, D), jnp.float32),
            pltpu.VMEM((T, D), jnp.float32),
            pltpu.VMEM((T, D), jnp.float32),
            pltpu.SemaphoreType.DMA,
            pltpu.SemaphoreType.DMA,
            pltpu.SemaphoreType.DMA,
            pltpu.SemaphoreType.DMA,
        ],
    )


def kernel(hidden_states, expert_outputs, gate_w, gate_b):
    wb, fi = _gating(hidden_states, gate_w.T, gate_b.reshape(1, E))
    eo_flat = expert_outputs.reshape(E * N, D)
    return _make_combine()(eo_flat, fi.reshape(N * K), wb.reshape(N * 2 * L))


# speedup vs baseline: 3.3788x; 1.0173x over previous
import functools

import jax
import jax.numpy as jnp
from jax import lax
from jax.experimental import pallas as pl
from jax.experimental.pallas import tpu as pltpu
from jax.experimental.pallas import tpu_sc as plsc

E = 8
N = 4096
D = 2048
K = 2
L = 16

NC = 2
NS = 16
NW = NC * NS
TPW = N // NW
T = 8
NSUB = TPW // T
ROWS = K * T

TOK_BLK = 512


def _gating_body(h_ref, gw_ref, gb_ref, wb_ref, fi_ref):
    blk = pl.program_id(0)
    h = h_ref[...]
    gw = gw_ref[...]
    gb = gb_ref[...]
    scores = lax.dot_general(h, gw, (((1,), (0,)), ((), ())),
                             preferred_element_type=jnp.float32) + gb
    iota_e = lax.broadcasted_iota(jnp.int32, (TOK_BLK, E), 1)
    m0 = jnp.max(scores, axis=1, keepdims=True)
    a0 = jnp.min(jnp.where(scores == m0, iota_e, E), axis=1, keepdims=True)
    masked = jnp.where(iota_e == a0, -jnp.inf, scores)
    m1 = jnp.max(masked, axis=1, keepdims=True)
    a1 = jnp.min(jnp.where(masked == m1, iota_e, E), axis=1, keepdims=True)
    w0 = 1.0 / (1.0 + jnp.exp(m1 - m0))
    w1 = 1.0 - w0
    lane = lax.broadcasted_iota(jnp.int32, (TOK_BLK, 2 * L), 1)
    wb_ref[...] = jnp.where(lane < L, w0, w1)
    tok = blk * TOK_BLK + lax.broadcasted_iota(jnp.int32, (TOK_BLK, K), 0)
    which = lax.broadcasted_iota(jnp.int32, (TOK_BLK, K), 1)
    fi_ref[...] = jnp.where(which == 0, a0, a1) * N + tok


def _gating(hidden_states, gate_w, gate_b2d):
    return pl.pallas_call(
        _gating_body,
        grid=(N // TOK_BLK,),
        in_specs=[
            pl.BlockSpec((TOK_BLK, D), lambda i: (i, 0)),
            pl.BlockSpec((D, E), lambda i: (0, 0)),
            pl.BlockSpec((1, E), lambda i: (0, 0)),
        ],
        out_specs=[
            pl.BlockSpec((TOK_BLK, 2 * L), lambda i: (i, 0)),
            pl.BlockSpec((TOK_BLK, K), lambda i: (i, 0)),
        ],
        out_shape=[
            jax.ShapeDtypeStruct((N, 2 * L), jnp.float32),
            jax.ShapeDtypeStruct((N, K), jnp.int32),
        ],
    )(hidden_states, gate_w, gate_b2d)


def _combine_body(eo_ref, idx_ref, w_ref, out_ref,
                  idx_v, w_v, r0, r1, o0, o1, g0, g1, s0, s1):
    wid = lax.axis_index("s") * NC + lax.axis_index("c")
    tok0 = wid * TPW
    pltpu.sync_copy(idx_ref.at[pl.ds(tok0 * K, TPW * K)], idx_v)
    pltpu.sync_copy(w_ref.at[pl.ds(tok0 * 2 * L, TPW * 2 * L)], w_v)

    rbufs = (r0, r1)
    obufs = (o0, o1)
    gsems = (g0, g1)
    ssems = (s0, s1)

    def start_gather(j, b):
        return pltpu.async_copy(
            eo_ref.at[idx_v.at[pl.ds(j * ROWS, ROWS)]], rbufs[b], gsems[b])

    def wait_gather(j, b):
        pltpu.make_async_copy(
            eo_ref.at[idx_v.at[pl.ds(j * ROWS, ROWS)]], rbufs[b], gsems[b]).wait()

    def start_store(j, b):
        return pltpu.async_copy(
            obufs[b], out_ref.at[pl.ds(tok0 + j * T, T)], ssems[b])

    def wait_store(j, b):
        pltpu.make_async_copy(
            obufs[b], out_ref.at[pl.ds(tok0 + j * T, T)], ssems[b]).wait()

    start_gather(0, 0)

    def pair_body(gp, _):
        for b in (0, 1):
            j = 2 * gp + b
            if b == 0:
                start_gather(j + 1, 1)
            else:
                @pl.when(gp < NSUB // 2 - 1)
                def _():
                    start_gather(j + 1, 0)
            wait_gather(j, b)

            @pl.when(gp > 0)
            def _():
                wait_store(j, b)

            def tok_body(t, _, jj=j, bb=b):
                woff = (jj * T + t) * (2 * L)
                wv0 = w_v[pl.ds(woff, L)]
                wv1 = w_v[pl.ds(woff + L, L)]

                @plsc.parallel_loop(0, D, L, unroll=8)
                def d_body(c):
                    obufs[bb][t, pl.ds(c, L)] = (
                        rbufs[bb][2 * t, pl.ds(c, L)] * wv0
                        + rbufs[bb][2 * t + 1, pl.ds(c, L)] * wv1)

                return 0

            lax.fori_loop(0, T, tok_body, 0)
            start_store(j, b)
        return 0

    lax.fori_loop(0, NSUB // 2, pair_body, 0)
    for b in (0, 1):
        wait_store(NSUB - 2 + b, b)


@functools.cache
def _make_combine():
    return pl.kernel(
        _combine_body,
        out_type=jax.ShapeDtypeStruct((N, D), jnp.float32),
        mesh=plsc.VectorSubcoreMesh(core_axis_name="c", subcore_axis_name="s",
                                    num_cores=NC, num_subcores=NS),
        scratch_types=[
            pltpu.VMEM((TPW * K,), jnp.int32),
            pltpu.VMEM((TPW * 2 * L,), jnp.float32),
            pltpu.VMEM((ROWS, D), jnp.float32),
            pltpu.VMEM((ROWS, D), jnp.float32),
            pltpu.VMEM((T, D), jnp.float32),
            pltpu.VMEM((T, D), jnp.float32),
            pltpu.SemaphoreType.DMA,
            pltpu.SemaphoreType.DMA,
            pltpu.SemaphoreType.DMA,
            pltpu.SemaphoreType.DMA,
        ],
    )


def kernel(hidden_states, expert_outputs, gate_w, gate_b):
    wb, fi = _gating(hidden_states, gate_w.T, gate_b.reshape(1, E))
    eo_flat = expert_outputs.reshape(E * N, D)
    return _make_combine()(eo_flat, fi.reshape(N * K), wb.reshape(N * 2 * L))


# speedup vs baseline: 3.4448x; 1.0195x over previous
import functools

import jax
import jax.numpy as jnp
from jax import lax
from jax.experimental import pallas as pl
from jax.experimental.pallas import tpu as pltpu
from jax.experimental.pallas import tpu_sc as plsc

E = 8
N = 4096
D = 2048
K = 2
L = 16

NC = 2
NS = 16
NW = NC * NS
TPW = N // NW
T = 4
NSUB = TPW // T
ROWS = K * T

TOK_BLK = 512


def _gating_body(h_ref, gw_ref, gb_ref, wb_ref, fi_ref):
    blk = pl.program_id(0)
    h = h_ref[...]
    gw = gw_ref[...]
    gb = gb_ref[...]
    scores = lax.dot_general(h, gw, (((1,), (0,)), ((), ())),
                             preferred_element_type=jnp.float32) + gb
    iota_e = lax.broadcasted_iota(jnp.int32, (TOK_BLK, E), 1)
    m0 = jnp.max(scores, axis=1, keepdims=True)
    a0 = jnp.min(jnp.where(scores == m0, iota_e, E), axis=1, keepdims=True)
    masked = jnp.where(iota_e == a0, -jnp.inf, scores)
    m1 = jnp.max(masked, axis=1, keepdims=True)
    a1 = jnp.min(jnp.where(masked == m1, iota_e, E), axis=1, keepdims=True)
    w0 = 1.0 / (1.0 + jnp.exp(m1 - m0))
    w1 = 1.0 - w0
    lane = lax.broadcasted_iota(jnp.int32, (TOK_BLK, 2 * L), 1)
    wb_ref[...] = jnp.where(lane < L, w0, w1)
    tok = blk * TOK_BLK + lax.broadcasted_iota(jnp.int32, (TOK_BLK, K), 0)
    which = lax.broadcasted_iota(jnp.int32, (TOK_BLK, K), 1)
    fi_ref[...] = jnp.where(which == 0, a0, a1) * N + tok


def _gating(hidden_states, gate_w, gate_b2d):
    return pl.pallas_call(
        _gating_body,
        grid=(N // TOK_BLK,),
        in_specs=[
            pl.BlockSpec((TOK_BLK, D), lambda i: (i, 0)),
            pl.BlockSpec((D, E), lambda i: (0, 0)),
            pl.BlockSpec((1, E), lambda i: (0, 0)),
        ],
        out_specs=[
            pl.BlockSpec((TOK_BLK, 2 * L), lambda i: (i, 0)),
            pl.BlockSpec((TOK_BLK, K), lambda i: (i, 0)),
        ],
        out_shape=[
            jax.ShapeDtypeStruct((N, 2 * L), jnp.float32),
            jax.ShapeDtypeStruct((N, K), jnp.int32),
        ],
    )(hidden_states, gate_w, gate_b2d)


def _combine_body(eo_ref, idx_ref, w_ref, out_ref,
                  idx_v, w_v, r0, r1, r2, r3, o0, o1, g0, g1, g2, g3, s0, s1):
    wid = lax.axis_index("s") * NC + lax.axis_index("c")
    tok0 = wid * TPW
    pltpu.sync_copy(idx_ref.at[pl.ds(tok0 * K, TPW * K)], idx_v)
    pltpu.sync_copy(w_ref.at[pl.ds(tok0 * 2 * L, TPW * 2 * L)], w_v)

    rbufs = (r0, r1, r2, r3)
    obufs = (o0, o1)
    gsems = (g0, g1, g2, g3)
    ssems = (s0, s1)

    def start_gather(j, b):
        return pltpu.async_copy(
            eo_ref.at[idx_v.at[pl.ds(j * ROWS, ROWS)]], rbufs[b], gsems[b])

    def wait_gather(j, b):
        pltpu.make_async_copy(
            eo_ref.at[idx_v.at[pl.ds(j * ROWS, ROWS)]], rbufs[b], gsems[b]).wait()

    def start_store(j, b):
        return pltpu.async_copy(
            obufs[b], out_ref.at[pl.ds(tok0 + j * T, T)], ssems[b])

    def wait_store(j, b):
        pltpu.make_async_copy(
            obufs[b], out_ref.at[pl.ds(tok0 + j * T, T)], ssems[b]).wait()

    for jj in range(3):
        start_gather(jj, jj)

    def quad_body(gp, _):
        for b in range(4):
            j = 4 * gp + b

            @pl.when(j + 3 < NSUB)
            def _():
                start_gather(j + 3, (b + 3) % 4)

            wait_gather(j, b)

            ob = b % 2

            @pl.when(j >= 2)
            def _():
                wait_store(j, ob)

            def tok_body(t, _, jj=j, bb=b, oo=ob):
                woff = (jj * T + t) * (2 * L)
                wv0 = w_v[pl.ds(woff, L)]
                wv1 = w_v[pl.ds(woff + L, L)]

                @plsc.parallel_loop(0, D, L, unroll=8)
                def d_body(c):
                    obufs[oo][t, pl.ds(c, L)] = (
                        rbufs[bb][2 * t, pl.ds(c, L)] * wv0
                        + rbufs[bb][2 * t + 1, pl.ds(c, L)] * wv1)

                return 0

            lax.fori_loop(0, T, tok_body, 0)
            start_store(j, ob)
        return 0

    lax.fori_loop(0, NSUB // 4, quad_body, 0)
    for b in (0, 1):
        wait_store(NSUB - 2 + b, b)


@functools.cache
def _make_combine():
    return pl.kernel(
        _combine_body,
        out_type=jax.ShapeDtypeStruct((N, D), jnp.float32),
        mesh=plsc.VectorSubcoreMesh(core_axis_name="c", subcore_axis_name="s",
                                    num_cores=NC, num_subcores=NS),
        scratch_types=[
            pltpu.VMEM((TPW * K,), jnp.int32),
            pltpu.VMEM((TPW * 2 * L,), jnp.float32),
            pltpu.VMEM((ROWS, D), jnp.float32),
            pltpu.VMEM((ROWS, D), jnp.float32),
            pltpu.VMEM((ROWS, D), jnp.float32),
            pltpu.VMEM((ROWS, D), jnp.float32),
            pltpu.VMEM((T, D), jnp.float32),
            pltpu.VMEM((T, D), jnp.float32),
            pltpu.SemaphoreType.DMA,
            pltpu.SemaphoreType.DMA,
            pltpu.SemaphoreType.DMA,
            pltpu.SemaphoreType.DMA,
            pltpu.SemaphoreType.DMA,
            pltpu.SemaphoreType.DMA,
        ],
    )


def kernel(hidden_states, expert_outputs, gate_w, gate_b):
    wb, fi = _gating(hidden_states, gate_w.T, gate_b.reshape(1, E))
    eo_flat = expert_outputs.reshape(E * N, D)
    return _make_combine()(eo_flat, fi.reshape(N * K), wb.reshape(N * 2 * L))
